# baseline (speedup 1.0000x reference)
import functools

import jax
import jax.numpy as jnp
from jax.experimental import pallas as pl
from jax.experimental.pallas import tpu as pltpu
from jax.experimental.pallas import tpu_sc as plsc

B = 16384
F = 26
V = 100000
D = 32
ND = 13
H1 = 256
H2 = 128
EPS = 1e-5

BF = B * F
R = 512
NB = B // R
W_SC = 256
DG = 128


def _sc_gather_emb(emb4, idx4_t):
    mesh = plsc.VectorSubcoreMesh(core_axis_name="c", subcore_axis_name="s")

    @pl.kernel(
        out_type=jax.ShapeDtypeStruct((F, B, DG), emb4.dtype),
        mesh=mesh,
    )
    def gather_kernel(e_hbm, i_hbm, o_hbm):
        def body(i_vmem, o_vmem):
            pltpu.sync_copy(e_hbm.at[i_vmem.at[0, 0]], o_vmem.at[0])

        pltpu.emit_pipeline(
            body,
            grid=(B // W_SC, F),
            in_specs=[pl.BlockSpec((1, 1, W_SC), lambda i, f: (f, 0, i))],
            out_specs=[pl.BlockSpec((1, W_SC, DG), lambda i, f: (f, i, 0))],
            core_axis_name=("c", "s"),
            dimension_semantics=(pltpu.PARALLEL, pltpu.PARALLEL),
        )(i_hbm, o_hbm)

    return gather_kernel(emb4, idx4_t)


def _sc_gather_fm(fm_flat, idx_flat):
    n = idx_flat.shape[1]
    mesh = plsc.VectorSubcoreMesh(core_axis_name="c", subcore_axis_name="s")

    @pl.kernel(
        out_type=jax.ShapeDtypeStruct((n, 1), fm_flat.dtype),
        mesh=mesh,
        compiler_params=pltpu.CompilerParams(use_tc_tiling_on_sc=False),
    )
    def gather_kernel(f_hbm, i_hbm, o_hbm):
        def body(i_vmem, o_vmem):
            pltpu.sync_copy(f_hbm.at[i_vmem.at[0]], o_vmem)

        pltpu.emit_pipeline(
            body,
            grid=(n // W_SC,),
            in_specs=[pl.BlockSpec((1, W_SC), lambda i: (0, i))],
            out_specs=[pl.BlockSpec((W_SC, 1), lambda i: (i, 0))],
            core_axis_name=("c", "s"),
            dimension_semantics=(pltpu.PARALLEL,),
        )(i_hbm, o_hbm)

    return gather_kernel(fm_flat, idx_flat)


def _extract_fields(x4, km):
    out = []
    for f in range(F):
        row = x4[f]
        kf = km[f]
        xf = jnp.zeros((row.shape[0], D), dtype=jnp.float32)
        for g in range(4):
            xf = xf + jnp.where(kf == g, row[:, g * D:(g + 1) * D], 0.0)
        out.append(xf)
    return out


def _stats_body(xe_ref, km_ref, xn_ref, oe_ref, on_ref):
    i = pl.program_id(0)

    @pl.when(i == 0)
    def _():
        oe_ref[...] = jnp.zeros_like(oe_ref)
        on_ref[...] = jnp.zeros_like(on_ref)

    xe = jnp.concatenate(_extract_fields(xe_ref[...], km_ref[...]), axis=1)
    xn = xn_ref[...]
    oe_ref[0:1, :] += jnp.sum(xe, axis=0, keepdims=True)
    oe_ref[1:2, :] += jnp.sum(xe * xe, axis=0, keepdims=True)
    on_ref[0:1, :] += jnp.sum(xn, axis=0, keepdims=True)
    on_ref[1:2, :] += jnp.sum(xn * xn, axis=0, keepdims=True)


def _bn_coeffs(stats, g, be):
    mean = stats[0:1, :] * (1.0 / B)
    var = stats[1:2, :] * (1.0 / B) - mean * mean
    a = g * jax.lax.rsqrt(var + EPS)
    c = be - mean * a
    return a, c


def _h1_body(xe_ref, km_ref, xn_ref, se_ref, sn_ref, g0e_ref, be0e_ref,
             g0n_ref, be0n_ref, w1e_ref, w1n_ref, b1_ref, h1_ref, so_ref,
             st1_ref):
    i = pl.program_id(0)

    @pl.when(i == 0)
    def _():
        st1_ref[...] = jnp.zeros_like(st1_ref)

    fields = _extract_fields(xe_ref[...], km_ref[...])
    xe = jnp.concatenate(fields, axis=1)
    xn = xn_ref[...]

    s = jnp.zeros((xe.shape[0], D), dtype=jnp.float32)
    sq = jnp.zeros((xe.shape[0], D), dtype=jnp.float32)
    for sl in fields:
        s = s + sl
        sq = sq + sl * sl
    so_ref[...] = 0.5 * (s * s - sq)

    ae, ce = _bn_coeffs(se_ref[...], g0e_ref[...], be0e_ref[...])
    an, cn = _bn_coeffs(sn_ref[...], g0n_ref[...], be0n_ref[...])
    xen = xe * ae + ce
    xnn = xn * an + cn
    h1 = (jnp.dot(xen, w1e_ref[...], preferred_element_type=jnp.float32)
          + jnp.dot(xnn, w1n_ref[...], preferred_element_type=jnp.float32)
          + b1_ref[...])
    h1_ref[...] = h1
    st1_ref[0:1, :] += jnp.sum(h1, axis=0, keepdims=True)
    st1_ref[1:2, :] += jnp.sum(h1 * h1, axis=0, keepdims=True)


def _h2_body(h1_ref, st1_ref, g1_ref, be1_ref, w2_ref, b2_ref, h2_ref,
             st2_ref):
    i = pl.program_id(0)

    @pl.when(i == 0)
    def _():
        st2_ref[...] = jnp.zeros_like(st2_ref)

    a1, c1 = _bn_coeffs(st1_ref[...], g1_ref[...], be1_ref[...])
    h1n = h1_ref[...] * a1 + c1
    h2 = (jnp.dot(h1n, w2_ref[...], preferred_element_type=jnp.float32)
          + b2_ref[...])
    h2_ref[...] = h2
    st2_ref[0:1, :] += jnp.sum(h2, axis=0, keepdims=True)
    st2_ref[1:2, :] += jnp.sum(h2 * h2, axis=0, keepdims=True)


def _out_body(h2_ref, st2_ref, g2_ref, be2_ref, fo_ref, so_ref, wpf_ref,
              wps_ref, wpd_ref, bp_ref, out_ref):
    a2, c2 = _bn_coeffs(st2_ref[...], g2_ref[...], be2_ref[...])
    h2n = h2_ref[...] * a2 + c2
    val = (jnp.sum(h2n * wpd_ref[...], axis=1, keepdims=True)
           + jnp.sum(so_ref[...] * wps_ref[...], axis=1, keepdims=True)
           + jnp.sum(fo_ref[...] * wpf_ref[...], axis=1, keepdims=True)
           + bp_ref[...])
    out_ref[...] = jax.nn.sigmoid(val)


def _bcast_spec(shape):
    return pl.BlockSpec(shape, lambda i: (0, 0))


def _row_spec(width):
    return pl.BlockSpec((R, width), lambda i: (i, 0))


@jax.jit
def kernel(numb_features, features, emb_table, fm_table, W1, b1, W2, b2, Wp,
           bp, g0, be0, g1, be1, g2, be2):
    flat_idx = (features + (jnp.arange(F, dtype=jnp.int32) * V)[None, :])
    idx4_t = (flat_idx // 4).T.reshape(F, 1, B)
    km = (flat_idx % 4).T.reshape(F, B, 1)
    flat_idx = flat_idx.reshape(1, BF)

    emb4 = emb_table.reshape(F * V // 4, 4 * D)
    fm_flat = fm_table.reshape(F * V, 1)

    x4 = _sc_gather_emb(emb4, idx4_t)
    fo = _sc_gather_fm(fm_flat, flat_idx)
    fo = fo.reshape(B, F)

    x4_spec = pl.BlockSpec((F, R, DG), lambda i: (0, i, 0))
    km_spec = pl.BlockSpec((F, R, 1), lambda i: (0, i, 0))

    se, sn = pl.pallas_call(
        _stats_body,
        grid=(NB,),
        in_specs=[x4_spec, km_spec, _row_spec(ND)],
        out_specs=[_bcast_spec((8, F * D)), _bcast_spec((8, ND))],
        out_shape=[jax.ShapeDtypeStruct((8, F * D), jnp.float32),
                   jax.ShapeDtypeStruct((8, ND), jnp.float32)],
    )(x4, km, numb_features)

    h1, so, st1 = pl.pallas_call(
        _h1_body,
        grid=(NB,),
        in_specs=[
            x4_spec, km_spec, _row_spec(ND),
            _bcast_spec((8, F * D)), _bcast_spec((8, ND)),
            _bcast_spec((1, F * D)), _bcast_spec((1, F * D)),
            _bcast_spec((1, ND)), _bcast_spec((1, ND)),
            _bcast_spec((F * D, H1)), _bcast_spec((ND, H1)),
            _bcast_spec((1, H1)),
        ],
        out_specs=[_row_spec(H1), _row_spec(D), _bcast_spec((8, H1))],
        out_shape=[jax.ShapeDtypeStruct((B, H1), jnp.float32),
                   jax.ShapeDtypeStruct((B, D), jnp.float32),
                   jax.ShapeDtypeStruct((8, H1), jnp.float32)],
    )(x4, km, numb_features, se, sn,
      g0[:F * D].reshape(1, F * D), be0[:F * D].reshape(1, F * D),
      g0[F * D:].reshape(1, ND), be0[F * D:].reshape(1, ND),
      W1[:F * D], W1[F * D:], b1.reshape(1, H1))

    h2, st2 = pl.pallas_call(
        _h2_body,
        grid=(NB,),
        in_specs=[
            _row_spec(H1), _bcast_spec((8, H1)),
            _bcast_spec((1, H1)), _bcast_spec((1, H1)),
            _bcast_spec((H1, H2)), _bcast_spec((1, H2)),
        ],
        out_specs=[_row_spec(H2), _bcast_spec((8, H2))],
        out_shape=[jax.ShapeDtypeStruct((B, H2), jnp.float32),
                   jax.ShapeDtypeStruct((8, H2), jnp.float32)],
    )(h1, st1, g1.reshape(1, H1), be1.reshape(1, H1), W2, b2.reshape(1, H2))

    out = pl.pallas_call(
        _out_body,
        grid=(NB,),
        in_specs=[
            _row_spec(H2), _bcast_spec((8, H2)),
            _bcast_spec((1, H2)), _bcast_spec((1, H2)),
            _row_spec(F), _row_spec(D),
            _bcast_spec((1, F)), _bcast_spec((1, D)), _bcast_spec((1, H2)),
            _bcast_spec((1, 1)),
        ],
        out_specs=_row_spec(1),
        out_shape=jax.ShapeDtypeStruct((B, 1), jnp.float32),
    )(h2, st2, g2.reshape(1, H2), be2.reshape(1, H2), fo, so,
      Wp[:F].reshape(1, F), Wp[F:F + D].reshape(1, D),
      Wp[F + D:].reshape(1, H2), bp.reshape(1, 1))

    return out


# speedup vs baseline: 3.6876x; 3.6876x over previous
import jax
import jax.numpy as jnp
from jax.experimental import pallas as pl
from jax.experimental.pallas import tpu as pltpu
from jax.experimental.pallas import tpu_sc as plsc

B = 16384
F = 26
V = 100000
D = 32
ND = 13
H1 = 256
H2 = 128
EPS = 1e-5

BF = B * F
R = 1024
NB = B // R
W_SC = 512


def _sc_gather(emb_flat, fm32, idx_t, idx32_t):
    mesh = plsc.VectorSubcoreMesh(core_axis_name="c", subcore_axis_name="s")

    @pl.kernel(
        out_type=[jax.ShapeDtypeStruct((B, F * D), emb_flat.dtype),
                  jax.ShapeDtypeStruct((B, F * D), fm32.dtype)],
        mesh=mesh,
        compiler_params=pltpu.CompilerParams(use_tc_tiling_on_sc=False),
    )
    def gather_kernel(e_hbm, f_hbm, ie_hbm, if_hbm, oe_hbm, of_hbm):
        def body(ie_vmem, if_vmem, oe_vmem, of_vmem):
            pltpu.sync_copy(e_hbm.at[ie_vmem.at[0, 0]], oe_vmem)
            pltpu.sync_copy(f_hbm.at[if_vmem.at[0, 0]], of_vmem)

        pltpu.emit_pipeline(
            body,
            grid=(B // W_SC, F),
            in_specs=[pl.BlockSpec((1, 1, W_SC), lambda i, f: (f, 0, i)),
                      pl.BlockSpec((1, 1, W_SC), lambda i, f: (f, 0, i))],
            out_specs=[pl.BlockSpec((W_SC, D), lambda i, f: (i, f)),
                       pl.BlockSpec((W_SC, D), lambda i, f: (i, f))],
            core_axis_name=("c", "s"),
            dimension_semantics=(pltpu.PARALLEL, pltpu.PARALLEL),
        )(ie_hbm, if_hbm, oe_hbm, of_hbm)

    return gather_kernel(emb_flat, fm32, idx_t, idx32_t)


def _stats_body(xe_ref, xn_ref, oe_ref, on_ref):
    i = pl.program_id(0)

    @pl.when(i == 0)
    def _():
        oe_ref[...] = jnp.zeros_like(oe_ref)
        on_ref[...] = jnp.zeros_like(on_ref)

    xe = xe_ref[...]
    xn = xn_ref[...]
    oe_ref[0:1, :] += jnp.sum(xe, axis=0, keepdims=True)
    oe_ref[1:2, :] += jnp.sum(xe * xe, axis=0, keepdims=True)
    on_ref[0:1, :] += jnp.sum(xn, axis=0, keepdims=True)
    on_ref[1:2, :] += jnp.sum(xn * xn, axis=0, keepdims=True)


def _bn_coeffs(stats, g, be):
    mean = stats[0:1, :] * (1.0 / B)
    var = stats[1:2, :] * (1.0 / B) - mean * mean
    a = g * jax.lax.rsqrt(var + EPS)
    c = be - mean * a
    return a, c


def _h1_body(xe_ref, xn_ref, se_ref, sn_ref, g0e_ref, be0e_ref, g0n_ref,
             be0n_ref, w1e_ref, w1n_ref, b1_ref, h1_ref, so_ref, st1_ref):
    i = pl.program_id(0)

    @pl.when(i == 0)
    def _():
        st1_ref[...] = jnp.zeros_like(st1_ref)

    xe = xe_ref[...]
    xn = xn_ref[...]

    s = jnp.zeros((xe.shape[0], D), dtype=jnp.float32)
    sq = jnp.zeros((xe.shape[0], D), dtype=jnp.float32)
    for f in range(F):
        sl = xe[:, f * D:(f + 1) * D]
        s = s + sl
        sq = sq + sl * sl
    so_ref[...] = 0.5 * (s * s - sq)

    ae, ce = _bn_coeffs(se_ref[...], g0e_ref[...], be0e_ref[...])
    an, cn = _bn_coeffs(sn_ref[...], g0n_ref[...], be0n_ref[...])
    xen = xe * ae + ce
    xnn = xn * an + cn
    h1 = (jnp.dot(xen, w1e_ref[...], preferred_element_type=jnp.float32)
          + jnp.dot(xnn, w1n_ref[...], preferred_element_type=jnp.float32)
          + b1_ref[...])
    h1_ref[...] = h1
    st1_ref[0:1, :] += jnp.sum(h1, axis=0, keepdims=True)
    st1_ref[1:2, :] += jnp.sum(h1 * h1, axis=0, keepdims=True)


def _h2_body(h1_ref, st1_ref, g1_ref, be1_ref, w2_ref, b2_ref, h2_ref,
             st2_ref):
    i = pl.program_id(0)

    @pl.when(i == 0)
    def _():
        st2_ref[...] = jnp.zeros_like(st2_ref)

    a1, c1 = _bn_coeffs(st1_ref[...], g1_ref[...], be1_ref[...])
    h1n = h1_ref[...] * a1 + c1
    h2 = (jnp.dot(h1n, w2_ref[...], preferred_element_type=jnp.float32)
          + b2_ref[...])
    h2_ref[...] = h2
    st2_ref[0:1, :] += jnp.sum(h2, axis=0, keepdims=True)
    st2_ref[1:2, :] += jnp.sum(h2 * h2, axis=0, keepdims=True)


def _out_body(h2_ref, st2_ref, g2_ref, be2_ref, fx_ref, km_ref, so_ref,
              wpf_ref, wps_ref, wpd_ref, bp_ref, out_ref):
    a2, c2 = _bn_coeffs(st2_ref[...], g2_ref[...], be2_ref[...])
    h2n = h2_ref[...] * a2 + c2
    val = (jnp.sum(h2n * wpd_ref[...], axis=1, keepdims=True)
           + jnp.sum(so_ref[...] * wps_ref[...], axis=1, keepdims=True)
           + bp_ref[...])
    fx = fx_ref[...]
    km = km_ref[...]
    lane = jax.lax.broadcasted_iota(jnp.int32, (1, D), 1)
    for f in range(F):
        sel = (km[:, f:f + 1] == lane).astype(jnp.float32)
        fo_f = jnp.sum(fx[:, f * D:(f + 1) * D] * sel, axis=1, keepdims=True)
        val = val + fo_f * wpf_ref[0, f]
    out_ref[...] = jax.nn.sigmoid(val)


def _bcast_spec(shape):
    return pl.BlockSpec(shape, lambda i: (0, 0))


def _row_spec(width):
    return pl.BlockSpec((R, width), lambda i: (i, 0))


@jax.jit
def kernel(numb_features, features, emb_table, fm_table, W1, b1, W2, b2, Wp,
           bp, g0, be0, g1, be1, g2, be2):
    foffs = (jnp.arange(F, dtype=jnp.int32))[None, :]
    flat_idx = features + foffs * V
    idx32 = foffs * (V // D) + features // D
    km = features % D

    idx_t = flat_idx.T.reshape(F, 1, B)
    idx32_t = idx32.T.reshape(F, 1, B)

    emb_flat = emb_table.reshape(F * V, D)
    fm32 = fm_table.reshape(F * V // D, D)

    xe, fx = _sc_gather(emb_flat, fm32, idx_t, idx32_t)

    se, sn = pl.pallas_call(
        _stats_body,
        grid=(NB,),
        in_specs=[_row_spec(F * D), _row_spec(ND)],
        out_specs=[_bcast_spec((8, F * D)), _bcast_spec((8, ND))],
        out_shape=[jax.ShapeDtypeStruct((8, F * D), jnp.float32),
                   jax.ShapeDtypeStruct((8, ND), jnp.float32)],
    )(xe, numb_features)

    h1, so, st1 = pl.pallas_call(
        _h1_body,
        grid=(NB,),
        in_specs=[
            _row_spec(F * D), _row_spec(ND),
            _bcast_spec((8, F * D)), _bcast_spec((8, ND)),
            _bcast_spec((1, F * D)), _bcast_spec((1, F * D)),
            _bcast_spec((1, ND)), _bcast_spec((1, ND)),
            _bcast_spec((F * D, H1)), _bcast_spec((ND, H1)),
            _bcast_spec((1, H1)),
        ],
        out_specs=[_row_spec(H1), _row_spec(D), _bcast_spec((8, H1))],
        out_shape=[jax.ShapeDtypeStruct((B, H1), jnp.float32),
                   jax.ShapeDtypeStruct((B, D), jnp.float32),
                   jax.ShapeDtypeStruct((8, H1), jnp.float32)],
    )(xe, numb_features, se, sn,
      g0[:F * D].reshape(1, F * D), be0[:F * D].reshape(1, F * D),
      g0[F * D:].reshape(1, ND), be0[F * D:].reshape(1, ND),
      W1[:F * D], W1[F * D:], b1.reshape(1, H1))

    h2, st2 = pl.pallas_call(
        _h2_body,
        grid=(NB,),
        in_specs=[
            _row_spec(H1), _bcast_spec((8, H1)),
            _bcast_spec((1, H1)), _bcast_spec((1, H1)),
            _bcast_spec((H1, H2)), _bcast_spec((1, H2)),
        ],
        out_specs=[_row_spec(H2), _bcast_spec((8, H2))],
        out_shape=[jax.ShapeDtypeStruct((B, H2), jnp.float32),
                   jax.ShapeDtypeStruct((8, H2), jnp.float32)],
    )(h1, st1, g1.reshape(1, H1), be1.reshape(1, H1), W2, b2.reshape(1, H2))

    out = pl.pallas_call(
        _out_body,
        grid=(NB,),
        in_specs=[
            _row_spec(H2), _bcast_spec((8, H2)),
            _bcast_spec((1, H2)), _bcast_spec((1, H2)),
            _row_spec(F * D), _row_spec(F), _row_spec(D),
            _bcast_spec((1, F)), _bcast_spec((1, D)), _bcast_spec((1, H2)),
            _bcast_spec((1, 1)),
        ],
        out_specs=_row_spec(1),
        out_shape=jax.ShapeDtypeStruct((B, 1), jnp.float32),
    )(h2, st2, g2.reshape(1, H2), be2.reshape(1, H2), fx, km, so,
      Wp[:F].reshape(1, F), Wp[F:F + D].reshape(1, D),
      Wp[F + D:].reshape(1, H2), bp.reshape(1, 1))

    return out


# speedup vs baseline: 4.0395x; 1.0954x over previous
import jax
import jax.numpy as jnp
from jax.experimental import pallas as pl
from jax.experimental.pallas import tpu as pltpu
from jax.experimental.pallas import tpu_sc as plsc

B = 16384
F = 26
V = 100000
D = 32
ND = 13
H1 = 256
H2 = 128
EPS = 1e-5

BF = B * F
R = 1024
NB = B // R
W_SC = 512


def _sc_gather(emb_flat, fm32, idx_t, idx32_t):
    mesh = plsc.VectorSubcoreMesh(core_axis_name="c", subcore_axis_name="s")

    @pl.kernel(
        out_type=[jax.ShapeDtypeStruct((B, F * D), emb_flat.dtype),
                  jax.ShapeDtypeStruct((B, F * D), fm32.dtype)],
        mesh=mesh,
        compiler_params=pltpu.CompilerParams(use_tc_tiling_on_sc=False),
    )
    def gather_kernel(e_hbm, f_hbm, ie_hbm, if_hbm, oe_hbm, of_hbm):
        def body(ie_vmem, if_vmem, oe_vmem, of_vmem):
            pltpu.sync_copy(e_hbm.at[ie_vmem.at[0, 0]], oe_vmem)
            pltpu.sync_copy(f_hbm.at[if_vmem.at[0, 0]], of_vmem)

        pltpu.emit_pipeline(
            body,
            grid=(B // W_SC, F),
            in_specs=[pl.BlockSpec((1, 1, W_SC), lambda i, f: (f, 0, i)),
                      pl.BlockSpec((1, 1, W_SC), lambda i, f: (f, 0, i))],
            out_specs=[pl.BlockSpec((W_SC, D), lambda i, f: (i, f)),
                       pl.BlockSpec((W_SC, D), lambda i, f: (i, f))],
            core_axis_name=("c", "s"),
            dimension_semantics=(pltpu.PARALLEL, pltpu.PARALLEL),
        )(ie_hbm, if_hbm, oe_hbm, of_hbm)

    return gather_kernel(emb_flat, fm32, idx_t, idx32_t)


def _stats_body(xe_ref, xn_ref, oe_ref, on_ref):
    i = pl.program_id(0)

    @pl.when(i == 0)
    def _():
        oe_ref[...] = jnp.zeros_like(oe_ref)
        on_ref[...] = jnp.zeros_like(on_ref)

    xe = xe_ref[...]
    xn = xn_ref[...]
    oe_ref[0:1, :] += jnp.sum(xe, axis=0, keepdims=True)
    oe_ref[1:2, :] += jnp.sum(xe * xe, axis=0, keepdims=True)
    on_ref[0:1, :] += jnp.sum(xn, axis=0, keepdims=True)
    on_ref[1:2, :] += jnp.sum(xn * xn, axis=0, keepdims=True)


def _bn_coeffs(stats, g, be):
    mean = stats[0:1, :] * (1.0 / B)
    var = stats[1:2, :] * (1.0 / B) - mean * mean
    a = g * jax.lax.rsqrt(var + EPS)
    c = be - mean * a
    return a, c


def _h1_body(xe_ref, xn_ref, se_ref, sn_ref, g0e_ref, be0e_ref, g0n_ref,
             be0n_ref, w1e_ref, w1n_ref, b1_ref, h1_ref, so_ref, st1_ref):
    i = pl.program_id(0)

    @pl.when(i == 0)
    def _():
        st1_ref[...] = jnp.zeros_like(st1_ref)

    xe = xe_ref[...]
    xn = xn_ref[...]

    s = jnp.zeros((xe.shape[0], D), dtype=jnp.float32)
    sq = jnp.zeros((xe.shape[0], D), dtype=jnp.float32)
    for f in range(F):
        sl = xe[:, f * D:(f + 1) * D]
        s = s + sl
        sq = sq + sl * sl
    so_ref[...] = 0.5 * (s * s - sq)

    ae, ce = _bn_coeffs(se_ref[...], g0e_ref[...], be0e_ref[...])
    an, cn = _bn_coeffs(sn_ref[...], g0n_ref[...], be0n_ref[...])
    xen = xe * ae + ce
    xnn = xn * an + cn
    h1 = (jnp.dot(xen, w1e_ref[...], preferred_element_type=jnp.float32)
          + jnp.dot(xnn, w1n_ref[...], preferred_element_type=jnp.float32)
          + b1_ref[...])
    h1_ref[...] = h1
    st1_ref[0:1, :] += jnp.sum(h1, axis=0, keepdims=True)
    st1_ref[1:2, :] += jnp.sum(h1 * h1, axis=0, keepdims=True)


def _h2_body(h1_ref, st1_ref, g1_ref, be1_ref, w2_ref, b2_ref, h2_ref,
             st2_ref):
    i = pl.program_id(0)

    @pl.when(i == 0)
    def _():
        st2_ref[...] = jnp.zeros_like(st2_ref)

    a1, c1 = _bn_coeffs(st1_ref[...], g1_ref[...], be1_ref[...])
    h1n = h1_ref[...] * a1 + c1
    h2 = (jnp.dot(h1n, w2_ref[...], preferred_element_type=jnp.float32)
          + b2_ref[...])
    h2_ref[...] = h2
    st2_ref[0:1, :] += jnp.sum(h2, axis=0, keepdims=True)
    st2_ref[1:2, :] += jnp.sum(h2 * h2, axis=0, keepdims=True)


def _out_body(h2_ref, st2_ref, g2_ref, be2_ref, fx_ref, km_ref, so_ref,
              wpfx_ref, wps_ref, wpd_ref, bp_ref, out_ref):
    a2, c2 = _bn_coeffs(st2_ref[...], g2_ref[...], be2_ref[...])
    h2n = h2_ref[...] * a2 + c2
    km = km_ref[...]
    kmx = jnp.concatenate(
        [jnp.broadcast_to(km[:, f:f + 1], (km.shape[0], D)) for f in range(F)],
        axis=1)
    lane = jax.lax.broadcasted_iota(jnp.int32, (1, F * D), 1) % D
    sel = (kmx == lane).astype(jnp.float32)
    val = (jnp.sum(h2n * wpd_ref[...], axis=1, keepdims=True)
           + jnp.sum(so_ref[...] * wps_ref[...], axis=1, keepdims=True)
           + jnp.sum(fx_ref[...] * sel * wpfx_ref[...], axis=1, keepdims=True)
           + bp_ref[...])
    out_ref[...] = jax.nn.sigmoid(val)


def _bcast_spec(shape):
    return pl.BlockSpec(shape, lambda i: (0, 0))


def _row_spec(width):
    return pl.BlockSpec((R, width), lambda i: (i, 0))


@jax.jit
def kernel(numb_features, features, emb_table, fm_table, W1, b1, W2, b2, Wp,
           bp, g0, be0, g1, be1, g2, be2):
    foffs = (jnp.arange(F, dtype=jnp.int32))[None, :]
    flat_idx = features + foffs * V
    idx32 = foffs * (V // D) + features // D
    km = features % D

    idx_t = flat_idx.T.reshape(F, 1, B)
    idx32_t = idx32.T.reshape(F, 1, B)

    emb_flat = emb_table.reshape(F * V, D)
    fm32 = fm_table.reshape(F * V // D, D)

    xe, fx = _sc_gather(emb_flat, fm32, idx_t, idx32_t)

    se, sn = pl.pallas_call(
        _stats_body,
        grid=(NB,),
        in_specs=[_row_spec(F * D), _row_spec(ND)],
        out_specs=[_bcast_spec((8, F * D)), _bcast_spec((8, ND))],
        out_shape=[jax.ShapeDtypeStruct((8, F * D), jnp.float32),
                   jax.ShapeDtypeStruct((8, ND), jnp.float32)],
    )(xe, numb_features)

    h1, so, st1 = pl.pallas_call(
        _h1_body,
        grid=(NB,),
        in_specs=[
            _row_spec(F * D), _row_spec(ND),
            _bcast_spec((8, F * D)), _bcast_spec((8, ND)),
            _bcast_spec((1, F * D)), _bcast_spec((1, F * D)),
            _bcast_spec((1, ND)), _bcast_spec((1, ND)),
            _bcast_spec((F * D, H1)), _bcast_spec((ND, H1)),
            _bcast_spec((1, H1)),
        ],
        out_specs=[_row_spec(H1), _row_spec(D), _bcast_spec((8, H1))],
        out_shape=[jax.ShapeDtypeStruct((B, H1), jnp.float32),
                   jax.ShapeDtypeStruct((B, D), jnp.float32),
                   jax.ShapeDtypeStruct((8, H1), jnp.float32)],
    )(xe, numb_features, se, sn,
      g0[:F * D].reshape(1, F * D), be0[:F * D].reshape(1, F * D),
      g0[F * D:].reshape(1, ND), be0[F * D:].reshape(1, ND),
      W1[:F * D], W1[F * D:], b1.reshape(1, H1))

    h2, st2 = pl.pallas_call(
        _h2_body,
        grid=(NB,),
        in_specs=[
            _row_spec(H1), _bcast_spec((8, H1)),
            _bcast_spec((1, H1)), _bcast_spec((1, H1)),
            _bcast_spec((H1, H2)), _bcast_spec((1, H2)),
        ],
        out_specs=[_row_spec(H2), _bcast_spec((8, H2))],
        out_shape=[jax.ShapeDtypeStruct((B, H2), jnp.float32),
                   jax.ShapeDtypeStruct((8, H2), jnp.float32)],
    )(h1, st1, g1.reshape(1, H1), be1.reshape(1, H1), W2, b2.reshape(1, H2))

    out = pl.pallas_call(
        _out_body,
        grid=(NB,),
        in_specs=[
            _row_spec(H2), _bcast_spec((8, H2)),
            _bcast_spec((1, H2)), _bcast_spec((1, H2)),
            _row_spec(F * D), _row_spec(F), _row_spec(D),
            _bcast_spec((1, F * D)), _bcast_spec((1, D)), _bcast_spec((1, H2)),
            _bcast_spec((1, 1)),
        ],
        out_specs=_row_spec(1),
        out_shape=jax.ShapeDtypeStruct((B, 1), jnp.float32),
    )(h2, st2, g2.reshape(1, H2), be2.reshape(1, H2), fx, km, so,
      jnp.repeat(Wp[:F, 0], D).reshape(1, F * D), Wp[F:F + D].reshape(1, D),
      Wp[F + D:].reshape(1, H2), bp.reshape(1, 1))

    return out
